# baseline (speedup 1.0000x reference)
import functools

import jax
import jax.numpy as jnp
from jax import lax
from jax.experimental import pallas as pl
from jax.experimental.pallas import tpu as pltpu
from jax.experimental.pallas import tpu_sc as plsc

N = 10000
E = 320000
D_IN = 128
OUT = 64
HID = 2 * OUT

NC = 2
NS = 16
CH = 128
EROWS = 2560
EPAD = EROWS * CH
TRASH = N
NPAD = 10112
ZR = NPAD // NS
WR = 1000
NWB = N // WR
R_SPMM = EROWS // NS
def _ring(fh):
  return (6, 4, 2) if fh == 64 else (8, 6, 2)
R_DEG = EROWS // (NC * NS)

_mesh = functools.partial(
    plsc.VectorSubcoreMesh,
    core_axis_name="c", subcore_axis_name="s", num_cores=NC, num_subcores=NS)


def _deg_kernel():
  @functools.partial(
      pl.kernel,
      out_type=jax.ShapeDtypeStruct((NC, NPAD, 16), jnp.float32),
      mesh=_mesh(),
      compiler_params=pltpu.CompilerParams(use_tc_tiling_on_sc=False),
      scratch_types=[
          pltpu.VMEM((R_DEG, CH), jnp.int32),
          pltpu.VMEM((ZR, 16), jnp.float32),
          pltpu.VMEM_SHARED((NPAD, 16), jnp.float32),
      ],
  )
  def k(dst_hbm, out_hbm, dst_v, ones_v, acc):
    c = lax.axis_index("c")
    t = lax.axis_index("s")
    w = c * NS + t

    def fill(i, _):
      ones_v[i] = jnp.ones((16,), jnp.float32)
      return 0
    lax.fori_loop(0, ZR, fill, 0)
    pltpu.sync_copy(ones_v, acc.at[pl.ds(t * ZR, ZR)])
    pltpu.sync_copy(dst_hbm.at[pl.ds(w * R_DEG, R_DEG)], dst_v)
    plsc.subcore_barrier()

    def body(j, _):
      pltpu.sync_copy(ones_v.at[pl.ds(0, CH)], acc.at[dst_v.at[j]], add=True)
      return 0
    lax.fori_loop(0, R_DEG, body, 0)
    plsc.subcore_barrier()
    pltpu.sync_copy(acc.at[pl.ds(t * ZR, ZR)], out_hbm.at[c, pl.ds(t * ZR, ZR)])

  return k


def _spmm_kernel(fh):
  NBUF, GLEAD, SLAG = _ring(fh)

  @functools.partial(
      pl.kernel,
      out_type=(jax.ShapeDtypeStruct((N, fh), jnp.float32),
                jax.ShapeDtypeStruct((N, fh), jnp.float32)),
      mesh=_mesh(),
      compiler_params=pltpu.CompilerParams(use_tc_tiling_on_sc=False),
      scratch_types=[
          pltpu.VMEM((R_SPMM, CH), jnp.int32),
          pltpu.VMEM((R_SPMM, CH), jnp.int32),
          pltpu.VMEM((NBUF, CH, fh), jnp.float32),
          pltpu.VMEM_SHARED((NPAD, fh), jnp.float32),
          pltpu.SemaphoreType.DMA((NBUF,)),
          pltpu.SemaphoreType.DMA((NBUF,)),
      ],
  )
  def k(hl_hbm, hr_hbm, src_hbm, dst_hbm, outl_hbm, outr_hbm,
        src_v, dst_v, rows_v, acc, gsem, ssem):
    c = lax.axis_index("c")
    t = lax.axis_index("s")

    pltpu.sync_copy(src_hbm.at[pl.ds(t * R_SPMM, R_SPMM)], src_v)
    pltpu.sync_copy(dst_hbm.at[pl.ds(t * R_SPMM, R_SPMM)], dst_v)

    def init(h_hbm):
      pltpu.sync_copy(h_hbm.at[pl.ds(t * WR, WR)], acc.at[pl.ds(t * WR, WR)])
    pl.when((c == 0) & (t < NWB))(lambda: init(hl_hbm))
    pl.when((c == 1) & (t < NWB))(lambda: init(hr_hbm))
    plsc.subcore_barrier()

    def scatter_all(h_hbm):
      for b in range(GLEAD):
        pltpu.async_copy(h_hbm.at[src_v.at[b]], rows_v.at[b], gsem.at[b])

      def body(j, _):
        @pl.when(j >= SLAG)
        def _():
          jb = j - SLAG
          b = lax.rem(jb, NBUF)
          pltpu.make_async_copy(rows_v.at[b], acc.at[dst_v.at[jb]],
                                ssem.at[b]).wait()
        @pl.when(j + GLEAD < R_SPMM)
        def _():
          jg = j + GLEAD
          b = lax.rem(jg, NBUF)
          pltpu.async_copy(h_hbm.at[src_v.at[jg]], rows_v.at[b], gsem.at[b])
        b = lax.rem(j, NBUF)
        pltpu.make_async_copy(h_hbm.at[src_v.at[j]], rows_v.at[b],
                              gsem.at[b]).wait()
        pltpu.make_async_copy(rows_v.at[b], acc.at[dst_v.at[j]],
                              ssem.at[b]).start(add=True)
        return 0
      lax.fori_loop(0, R_SPMM, body, 0)

      for j in range(R_SPMM - SLAG, R_SPMM):
        b = j % NBUF
        pltpu.make_async_copy(rows_v.at[b], acc.at[dst_v.at[j]],
                              ssem.at[b]).wait()
    pl.when(c == 0)(lambda: scatter_all(hl_hbm))
    pl.when(c == 1)(lambda: scatter_all(hr_hbm))
    plsc.subcore_barrier()

    def writeback(o_hbm):
      pltpu.sync_copy(acc.at[pl.ds(t * WR, WR)], o_hbm.at[pl.ds(t * WR, WR)])
    pl.when((c == 0) & (t < NWB))(lambda: writeback(outl_hbm))
    pl.when((c == 1) & (t < NWB))(lambda: writeback(outr_hbm))

  return k


_deg = _deg_kernel()
_spmm64 = _spmm_kernel(OUT)
_spmm32 = _spmm_kernel(OUT // 2)


RB = 1000
_GRID = N // RB


def _rows(i):
  return (i, 0)


def _full(i):
  return (0, 0)


def _tc_call(body, out_shapes, in_specs, out_specs):
  return pl.pallas_call(
      body,
      grid=(_GRID,),
      in_specs=in_specs,
      out_specs=out_specs,
      out_shape=out_shapes,
  )


def _bs(shape, imap):
  return pl.BlockSpec(shape, imap)


def _stage_a(deg_ref, x_ref, w_ref, dinv_ref, hl_ref, hr_ref):
  deg = deg_ref[0, :, 0] + deg_ref[1, :, 0] - 1.0
  dinv = lax.rsqrt(deg)[:, None]
  dinv_ref[...] = dinv
  h = jnp.dot(x_ref[...] * dinv, w_ref[...],
              preferred_element_type=jnp.float32)
  hl_ref[...] = h[:, :OUT]
  hr_ref[...] = h[:, OUT:]


def _stage_b(sl_ref, sr_ref, dinv_ref, b_ref, w_ref, hl_ref, hr_ref):
  s = jnp.concatenate([sl_ref[...], sr_ref[...]], axis=1)
  dinv = dinv_ref[...]
  x1 = jnp.maximum(s * dinv + b_ref[...], 0.0)
  h = jnp.dot(x1 * dinv, w_ref[...], preferred_element_type=jnp.float32)
  hl_ref[...] = h[:, :OUT // 2]
  hr_ref[...] = h[:, OUT // 2:]


def _stage_c(sl_ref, sr_ref, dinv_ref, b_ref, w_ref, x2_ref, hl_ref, hr_ref):
  s = jnp.concatenate([sl_ref[...], sr_ref[...]], axis=1)
  dinv = dinv_ref[...]
  x2 = s * dinv + b_ref[...]
  x2_ref[...] = x2
  h = jnp.dot(x2 * dinv, w_ref[...], preferred_element_type=jnp.float32)
  hl_ref[...] = h[:, :OUT // 2]
  hr_ref[...] = h[:, OUT // 2:]


def _stage_d(sl_ref, sr_ref, dinv_ref, b_ref, e_ref):
  s = jnp.concatenate([sl_ref[...], sr_ref[...]], axis=1)
  e_ref[...] = s * dinv_ref[...] + b_ref[...]


def kernel(x, edge_index, W1n, b1n, W2n, b2n, W1e, b1e, W2e, b2e):
  f32 = jnp.float32
  src = edge_index[0].astype(jnp.int32)
  dst = edge_index[1].astype(jnp.int32)
  pad = EPAD - E
  src2d = jnp.concatenate([src, jnp.zeros((pad,), jnp.int32)]).reshape(EROWS, CH)
  dst2d = jnp.concatenate([dst, jnp.full((pad,), TRASH, jnp.int32)]).reshape(EROWS, CH)

  deg2 = _deg(dst2d)

  sd = jax.ShapeDtypeStruct
  dinv, h1l, h1r = _tc_call(
      _stage_a,
      (sd((N, 1), f32), sd((N, OUT), f32), sd((N, OUT), f32)),
      [_bs((NC, RB, 16), lambda i: (0, i, 0)), _bs((RB, D_IN), _rows),
       _bs((D_IN, HID), _full)],
      [_bs((RB, 1), _rows), _bs((RB, OUT), _rows), _bs((RB, OUT), _rows)],
  )(deg2, x, W1n)

  s1l, s1r = _spmm64(h1l, h1r, src2d, dst2d)

  h2l, h2r = _tc_call(
      _stage_b,
      (sd((N, OUT // 2), f32), sd((N, OUT // 2), f32)),
      [_bs((RB, OUT), _rows), _bs((RB, OUT), _rows), _bs((RB, 1), _rows),
       _bs((1, HID), _full), _bs((HID, OUT), _full)],
      [_bs((RB, OUT // 2), _rows), _bs((RB, OUT // 2), _rows)],
  )(s1l, s1r, dinv, b1n.reshape(1, HID), W2n)

  s2l, s2r = _spmm32(h2l, h2r, src2d, dst2d)

  x2, h3l, h3r = _tc_call(
      _stage_c,
      (sd((N, OUT), f32), sd((N, OUT // 2), f32), sd((N, OUT // 2), f32)),
      [_bs((RB, OUT // 2), _rows), _bs((RB, OUT // 2), _rows),
       _bs((RB, 1), _rows), _bs((1, OUT), _full), _bs((OUT, OUT), _full)],
      [_bs((RB, OUT), _rows), _bs((RB, OUT // 2), _rows),
       _bs((RB, OUT // 2), _rows)],
  )(s2l, s2r, dinv, b2n.reshape(1, OUT), W2e)

  s3l, s3r = _spmm32(h3l, h3r, src2d, dst2d)

  e = _tc_call(
      _stage_d,
      sd((N, OUT), f32),
      [_bs((RB, OUT // 2), _rows), _bs((RB, OUT // 2), _rows),
       _bs((RB, 1), _rows), _bs((1, OUT), _full)],
      _bs((RB, OUT), _rows),
  )(s3l, s3r, dinv, b2e.reshape(1, OUT))

  return (x2, e)


# speedup vs baseline: 1.2651x; 1.2651x over previous
import functools

import jax
import jax.numpy as jnp
from jax import lax
from jax.experimental import pallas as pl
from jax.experimental.pallas import tpu as pltpu
from jax.experimental.pallas import tpu_sc as plsc

N = 10000
E = 320000
D_IN = 128
OUT = 64
HID = 2 * OUT

NC = 2
NS = 16
CH = 128
EROWS = 2560
EPAD = EROWS * CH
TRASH = N
NPAD = 10112
ZR = NPAD // NS
WR = 1000
NWB = N // WR
R_SPMM = EROWS // NS
def _ring(fh):
  return (6, 4, 2) if fh == 64 else (8, 6, 2)
R_DEG = EROWS // (NC * NS)

_mesh = functools.partial(
    plsc.VectorSubcoreMesh,
    core_axis_name="c", subcore_axis_name="s", num_cores=NC, num_subcores=NS)


def _deg_kernel():
  @functools.partial(
      pl.kernel,
      out_type=jax.ShapeDtypeStruct((NC, NPAD, 16), jnp.float32),
      mesh=_mesh(),
      compiler_params=pltpu.CompilerParams(use_tc_tiling_on_sc=False),
      scratch_types=[
          pltpu.VMEM((R_DEG, CH), jnp.int32),
          pltpu.VMEM((ZR, 16), jnp.float32),
          pltpu.VMEM_SHARED((NPAD, 16), jnp.float32),
      ],
  )
  def k(dst_hbm, out_hbm, dst_v, ones_v, acc):
    c = lax.axis_index("c")
    t = lax.axis_index("s")
    w = c * NS + t

    def fill(i, _):
      ones_v[i] = jnp.ones((16,), jnp.float32)
      return 0
    lax.fori_loop(0, ZR, fill, 0)
    pltpu.sync_copy(ones_v, acc.at[pl.ds(t * ZR, ZR)])
    pltpu.sync_copy(dst_hbm.at[pl.ds(w * R_DEG, R_DEG)], dst_v)
    plsc.subcore_barrier()

    def body(j, _):
      pltpu.sync_copy(ones_v.at[pl.ds(0, CH)], acc.at[dst_v.at[j]], add=True)
      return 0
    lax.fori_loop(0, R_DEG, body, 0)
    plsc.subcore_barrier()
    pltpu.sync_copy(acc.at[pl.ds(t * ZR, ZR)], out_hbm.at[c, pl.ds(t * ZR, ZR)])

  return k


def _spmm_kernel(fh, spmem_tab=False):
  NBUF, GLEAD, SLAG = _ring(fh)

  scratch = [
      pltpu.VMEM((R_SPMM, CH), jnp.int32),
      pltpu.VMEM((R_SPMM, CH), jnp.int32),
      pltpu.VMEM((NBUF, CH, fh), jnp.float32),
      pltpu.VMEM_SHARED((NPAD, fh), jnp.float32),
      pltpu.SemaphoreType.DMA((NBUF,)),
      pltpu.SemaphoreType.DMA((NBUF,)),
  ]
  if spmem_tab:
    scratch.append(pltpu.VMEM_SHARED((N, fh), jnp.float32))

  @functools.partial(
      pl.kernel,
      out_type=(jax.ShapeDtypeStruct((N, fh), jnp.float32),
                jax.ShapeDtypeStruct((N, fh), jnp.float32)),
      mesh=_mesh(),
      compiler_params=pltpu.CompilerParams(use_tc_tiling_on_sc=False),
      scratch_types=scratch,
  )
  def k(hl_hbm, hr_hbm, src_hbm, dst_hbm, outl_hbm, outr_hbm,
        src_v, dst_v, rows_v, acc, gsem, ssem, *maybe_tab):
    c = lax.axis_index("c")
    t = lax.axis_index("s")

    pltpu.sync_copy(src_hbm.at[pl.ds(t * R_SPMM, R_SPMM)], src_v)
    pltpu.sync_copy(dst_hbm.at[pl.ds(t * R_SPMM, R_SPMM)], dst_v)

    def init(h_hbm):
      pltpu.sync_copy(h_hbm.at[pl.ds(t * WR, WR)], acc.at[pl.ds(t * WR, WR)])
      if spmem_tab:
        pltpu.sync_copy(h_hbm.at[pl.ds(t * WR, WR)],
                        maybe_tab[0].at[pl.ds(t * WR, WR)])
    pl.when((c == 0) & (t < NWB))(lambda: init(hl_hbm))
    pl.when((c == 1) & (t < NWB))(lambda: init(hr_hbm))
    plsc.subcore_barrier()

    def scatter_all(h_hbm):
      tab = maybe_tab[0] if spmem_tab else h_hbm
      for b in range(GLEAD):
        pltpu.async_copy(tab.at[src_v.at[b]], rows_v.at[b], gsem.at[b])

      def body(j, _):
        @pl.when(j >= SLAG)
        def _():
          jb = j - SLAG
          b = lax.rem(jb, NBUF)
          pltpu.make_async_copy(rows_v.at[b], acc.at[dst_v.at[jb]],
                                ssem.at[b]).wait()
        @pl.when(j + GLEAD < R_SPMM)
        def _():
          jg = j + GLEAD
          b = lax.rem(jg, NBUF)
          pltpu.async_copy(tab.at[src_v.at[jg]], rows_v.at[b], gsem.at[b])
        b = lax.rem(j, NBUF)
        pltpu.make_async_copy(tab.at[src_v.at[j]], rows_v.at[b],
                              gsem.at[b]).wait()
        pltpu.make_async_copy(rows_v.at[b], acc.at[dst_v.at[j]],
                              ssem.at[b]).start(add=True)
        return 0
      lax.fori_loop(0, R_SPMM, body, 0)

      for j in range(R_SPMM - SLAG, R_SPMM):
        b = j % NBUF
        pltpu.make_async_copy(rows_v.at[b], acc.at[dst_v.at[j]],
                              ssem.at[b]).wait()
    pl.when(c == 0)(lambda: scatter_all(hl_hbm))
    pl.when(c == 1)(lambda: scatter_all(hr_hbm))
    plsc.subcore_barrier()

    def writeback(o_hbm):
      pltpu.sync_copy(acc.at[pl.ds(t * WR, WR)], o_hbm.at[pl.ds(t * WR, WR)])
    pl.when((c == 0) & (t < NWB))(lambda: writeback(outl_hbm))
    pl.when((c == 1) & (t < NWB))(lambda: writeback(outr_hbm))

  return k


_deg = _deg_kernel()
_spmm64 = _spmm_kernel(OUT)
_spmm32 = _spmm_kernel(OUT // 2, spmem_tab=True)


RB = 1000
_GRID = N // RB


def _rows(i):
  return (i, 0)


def _full(i):
  return (0, 0)


def _tc_call(body, out_shapes, in_specs, out_specs):
  return pl.pallas_call(
      body,
      grid=(_GRID,),
      in_specs=in_specs,
      out_specs=out_specs,
      out_shape=out_shapes,
  )


def _bs(shape, imap):
  return pl.BlockSpec(shape, imap)


def _stage_a(deg_ref, x_ref, w_ref, dinv_ref, hl_ref, hr_ref):
  deg = deg_ref[0, :, 0] + deg_ref[1, :, 0] - 1.0
  dinv = lax.rsqrt(deg)[:, None]
  dinv_ref[...] = dinv
  h = jnp.dot(x_ref[...] * dinv, w_ref[...],
              preferred_element_type=jnp.float32)
  hl_ref[...] = h[:, :OUT]
  hr_ref[...] = h[:, OUT:]


def _stage_b(sl_ref, sr_ref, dinv_ref, b_ref, w_ref, hl_ref, hr_ref):
  s = jnp.concatenate([sl_ref[...], sr_ref[...]], axis=1)
  dinv = dinv_ref[...]
  x1 = jnp.maximum(s * dinv + b_ref[...], 0.0)
  h = jnp.dot(x1 * dinv, w_ref[...], preferred_element_type=jnp.float32)
  hl_ref[...] = h[:, :OUT // 2]
  hr_ref[...] = h[:, OUT // 2:]


def _stage_c(sl_ref, sr_ref, dinv_ref, b_ref, w_ref, x2_ref, hl_ref, hr_ref):
  s = jnp.concatenate([sl_ref[...], sr_ref[...]], axis=1)
  dinv = dinv_ref[...]
  x2 = s * dinv + b_ref[...]
  x2_ref[...] = x2
  h = jnp.dot(x2 * dinv, w_ref[...], preferred_element_type=jnp.float32)
  hl_ref[...] = h[:, :OUT // 2]
  hr_ref[...] = h[:, OUT // 2:]


def _stage_d(sl_ref, sr_ref, dinv_ref, b_ref, e_ref):
  s = jnp.concatenate([sl_ref[...], sr_ref[...]], axis=1)
  e_ref[...] = s * dinv_ref[...] + b_ref[...]


def kernel(x, edge_index, W1n, b1n, W2n, b2n, W1e, b1e, W2e, b2e):
  f32 = jnp.float32
  src = edge_index[0].astype(jnp.int32)
  dst = edge_index[1].astype(jnp.int32)
  pad = EPAD - E
  src2d = jnp.concatenate([src, jnp.zeros((pad,), jnp.int32)]).reshape(EROWS, CH)
  dst2d = jnp.concatenate([dst, jnp.full((pad,), TRASH, jnp.int32)]).reshape(EROWS, CH)

  deg2 = _deg(dst2d)

  sd = jax.ShapeDtypeStruct
  dinv, h1l, h1r = _tc_call(
      _stage_a,
      (sd((N, 1), f32), sd((N, OUT), f32), sd((N, OUT), f32)),
      [_bs((NC, RB, 16), lambda i: (0, i, 0)), _bs((RB, D_IN), _rows),
       _bs((D_IN, HID), _full)],
      [_bs((RB, 1), _rows), _bs((RB, OUT), _rows), _bs((RB, OUT), _rows)],
  )(deg2, x, W1n)

  s1l, s1r = _spmm64(h1l, h1r, src2d, dst2d)

  h2l, h2r = _tc_call(
      _stage_b,
      (sd((N, OUT // 2), f32), sd((N, OUT // 2), f32)),
      [_bs((RB, OUT), _rows), _bs((RB, OUT), _rows), _bs((RB, 1), _rows),
       _bs((1, HID), _full), _bs((HID, OUT), _full)],
      [_bs((RB, OUT // 2), _rows), _bs((RB, OUT // 2), _rows)],
  )(s1l, s1r, dinv, b1n.reshape(1, HID), W2n)

  s2l, s2r = _spmm32(h2l, h2r, src2d, dst2d)

  x2, h3l, h3r = _tc_call(
      _stage_c,
      (sd((N, OUT), f32), sd((N, OUT // 2), f32), sd((N, OUT // 2), f32)),
      [_bs((RB, OUT // 2), _rows), _bs((RB, OUT // 2), _rows),
       _bs((RB, 1), _rows), _bs((1, OUT), _full), _bs((OUT, OUT), _full)],
      [_bs((RB, OUT), _rows), _bs((RB, OUT // 2), _rows),
       _bs((RB, OUT // 2), _rows)],
  )(s2l, s2r, dinv, b2n.reshape(1, OUT), W2e)

  s3l, s3r = _spmm32(h3l, h3r, src2d, dst2d)

  e = _tc_call(
      _stage_d,
      sd((N, OUT), f32),
      [_bs((RB, OUT // 2), _rows), _bs((RB, OUT // 2), _rows),
       _bs((RB, 1), _rows), _bs((1, OUT), _full)],
      _bs((RB, OUT), _rows),
  )(s3l, s3r, dinv, b2e.reshape(1, OUT))

  return (x2, e)


# speedup vs baseline: 1.6824x; 1.3299x over previous
import functools

import jax
import jax.numpy as jnp
from jax import lax
from jax.experimental import pallas as pl
from jax.experimental.pallas import tpu as pltpu
from jax.experimental.pallas import tpu_sc as plsc

N = 10000
E = 320000
D_IN = 128
OUT = 64
HID = 2 * OUT

NC = 2
NS = 16
CH = 128
EROWS = 2560
EPAD = EROWS * CH
TRASH = N
NPAD = 10112
ZR = NPAD // NS
WR = 1000
NWB = N // WR
R_SPMM = EROWS // NS
def _ring(fh):
  return (6, 4, 2) if fh == 64 else (8, 6, 2)
R_DEG = EROWS // (NC * NS)

_mesh = functools.partial(
    plsc.VectorSubcoreMesh,
    core_axis_name="c", subcore_axis_name="s", num_cores=NC, num_subcores=NS)


def _deg_kernel():
  @functools.partial(
      pl.kernel,
      out_type=jax.ShapeDtypeStruct((NC, NPAD, 16), jnp.float32),
      mesh=_mesh(),
      compiler_params=pltpu.CompilerParams(use_tc_tiling_on_sc=False),
      scratch_types=[
          pltpu.VMEM((R_DEG, CH), jnp.int32),
          pltpu.VMEM((ZR, 16), jnp.float32),
          pltpu.VMEM_SHARED((NPAD, 16), jnp.float32),
      ],
  )
  def k(dst_hbm, out_hbm, dst_v, ones_v, acc):
    c = lax.axis_index("c")
    t = lax.axis_index("s")
    w = c * NS + t

    def fill(i, _):
      ones_v[i] = jnp.ones((16,), jnp.float32)
      return 0
    lax.fori_loop(0, ZR, fill, 0)
    pltpu.sync_copy(ones_v, acc.at[pl.ds(t * ZR, ZR)])
    pltpu.sync_copy(dst_hbm.at[pl.ds(w * R_DEG, R_DEG)], dst_v)
    plsc.subcore_barrier()

    def body(j, _):
      pltpu.sync_copy(ones_v.at[pl.ds(0, CH)], acc.at[dst_v.at[j]], add=True)
      return 0
    lax.fori_loop(0, R_DEG, body, 0)
    plsc.subcore_barrier()
    pltpu.sync_copy(acc.at[pl.ds(t * ZR, ZR)], out_hbm.at[c, pl.ds(t * ZR, ZR)])

  return k


def _spmm_kernel(fh, spmem_tab=False):
  NBUF, GLEAD, SLAG = _ring(fh)

  scratch = [
      pltpu.VMEM((R_SPMM, CH), jnp.int32),
      pltpu.VMEM((R_SPMM, CH), jnp.int32),
      pltpu.VMEM((NBUF, CH, fh), jnp.float32),
      pltpu.VMEM_SHARED((NPAD, fh), jnp.float32),
      pltpu.SemaphoreType.DMA((NBUF,)),
      pltpu.SemaphoreType.DMA((NBUF,)),
  ]
  if spmem_tab:
    scratch.append(pltpu.VMEM_SHARED((N, fh), jnp.float32))

  @functools.partial(
      pl.kernel,
      out_type=(jax.ShapeDtypeStruct((N, fh), jnp.float32),
                jax.ShapeDtypeStruct((N, fh), jnp.float32)),
      mesh=_mesh(),
      compiler_params=pltpu.CompilerParams(use_tc_tiling_on_sc=False),
      scratch_types=scratch,
  )
  def k(hl_hbm, hr_hbm, src_hbm, dst_hbm, outl_hbm, outr_hbm,
        src_v, dst_v, rows_v, acc, gsem, ssem, *maybe_tab):
    c = lax.axis_index("c")
    t = lax.axis_index("s")

    pltpu.sync_copy(src_hbm.at[pl.ds(t * R_SPMM, R_SPMM)], src_v)
    pltpu.sync_copy(dst_hbm.at[pl.ds(t * R_SPMM, R_SPMM)], dst_v)

    def init(h_hbm):
      pltpu.sync_copy(h_hbm.at[pl.ds(t * WR, WR)], acc.at[pl.ds(t * WR, WR)])
      if spmem_tab:
        pltpu.sync_copy(h_hbm.at[pl.ds(t * WR, WR)],
                        maybe_tab[0].at[pl.ds(t * WR, WR)])
    pl.when((c == 0) & (t < NWB))(lambda: init(hl_hbm))
    pl.when((c == 1) & (t < NWB))(lambda: init(hr_hbm))
    plsc.subcore_barrier()

    def scatter_all(h_hbm):
      tab = maybe_tab[0] if spmem_tab else h_hbm
      for b in range(GLEAD):
        pltpu.async_copy(tab.at[src_v.at[b]], rows_v.at[b], gsem.at[b])

      def body(j, _):
        @pl.when(j >= SLAG)
        def _():
          jb = j - SLAG
          b = lax.rem(jb, NBUF)
          pltpu.make_async_copy(rows_v.at[b], acc.at[dst_v.at[jb]],
                                ssem.at[b]).wait()
        @pl.when(j + GLEAD < R_SPMM)
        def _():
          jg = j + GLEAD
          b = lax.rem(jg, NBUF)
          pltpu.async_copy(tab.at[src_v.at[jg]], rows_v.at[b], gsem.at[b])
        b = lax.rem(j, NBUF)
        pltpu.make_async_copy(tab.at[src_v.at[j]], rows_v.at[b],
                              gsem.at[b]).wait()
        pltpu.make_async_copy(rows_v.at[b], acc.at[dst_v.at[j]],
                              ssem.at[b]).start(add=True)
        return 0
      lax.fori_loop(0, R_SPMM, body, 0)

      for j in range(R_SPMM - SLAG, R_SPMM):
        b = j % NBUF
        pltpu.make_async_copy(rows_v.at[b], acc.at[dst_v.at[j]],
                              ssem.at[b]).wait()
    pl.when(c == 0)(lambda: scatter_all(hl_hbm))
    pl.when(c == 1)(lambda: scatter_all(hr_hbm))
    plsc.subcore_barrier()

    def writeback(o_hbm):
      pltpu.sync_copy(acc.at[pl.ds(t * WR, WR)], o_hbm.at[pl.ds(t * WR, WR)])
    pl.when((c == 0) & (t < NWB))(lambda: writeback(outl_hbm))
    pl.when((c == 1) & (t < NWB))(lambda: writeback(outr_hbm))

  return k


_deg = _deg_kernel()
_spmm32 = _spmm_kernel(OUT // 2, spmem_tab=True)


RB = 1000
_GRID = N // RB


def _rows(i):
  return (i, 0)


def _full(i):
  return (0, 0)


def _tc_call(body, out_shapes, in_specs, out_specs):
  return pl.pallas_call(
      body,
      grid=(_GRID,),
      in_specs=in_specs,
      out_specs=out_specs,
      out_shape=out_shapes,
  )


def _bs(shape, imap):
  return pl.BlockSpec(shape, imap)


def _stage_a(deg_ref, x_ref, w_ref, dinv_ref, h0_ref, h1_ref, h2_ref,
             h3_ref):
  deg = deg_ref[0, :, 0] + deg_ref[1, :, 0] - 1.0
  dinv = lax.rsqrt(deg)[:, None]
  dinv_ref[...] = dinv
  h = jnp.dot(x_ref[...] * dinv, w_ref[...],
              preferred_element_type=jnp.float32)
  q = OUT // 2
  h0_ref[...] = h[:, :q]
  h1_ref[...] = h[:, q:2 * q]
  h2_ref[...] = h[:, 2 * q:3 * q]
  h3_ref[...] = h[:, 3 * q:]


def _stage_b(s0_ref, s1_ref, s2_ref, s3_ref, dinv_ref, b_ref, w_ref,
             hl_ref, hr_ref):
  s = jnp.concatenate([s0_ref[...], s1_ref[...], s2_ref[...], s3_ref[...]],
                      axis=1)
  dinv = dinv_ref[...]
  x1 = jnp.maximum(s * dinv + b_ref[...], 0.0)
  h = jnp.dot(x1 * dinv, w_ref[...], preferred_element_type=jnp.float32)
  hl_ref[...] = h[:, :OUT // 2]
  hr_ref[...] = h[:, OUT // 2:]


def _stage_c(sl_ref, sr_ref, dinv_ref, b_ref, w_ref, x2_ref, hl_ref, hr_ref):
  s = jnp.concatenate([sl_ref[...], sr_ref[...]], axis=1)
  dinv = dinv_ref[...]
  x2 = s * dinv + b_ref[...]
  x2_ref[...] = x2
  h = jnp.dot(x2 * dinv, w_ref[...], preferred_element_type=jnp.float32)
  hl_ref[...] = h[:, :OUT // 2]
  hr_ref[...] = h[:, OUT // 2:]


def _stage_d(sl_ref, sr_ref, dinv_ref, b_ref, e_ref):
  s = jnp.concatenate([sl_ref[...], sr_ref[...]], axis=1)
  e_ref[...] = s * dinv_ref[...] + b_ref[...]


def kernel(x, edge_index, W1n, b1n, W2n, b2n, W1e, b1e, W2e, b2e):
  f32 = jnp.float32
  src = edge_index[0].astype(jnp.int32)
  dst = edge_index[1].astype(jnp.int32)
  pad = EPAD - E
  src2d = jnp.concatenate([src, jnp.zeros((pad,), jnp.int32)]).reshape(EROWS, CH)
  dst2d = jnp.concatenate([dst, jnp.full((pad,), TRASH, jnp.int32)]).reshape(EROWS, CH)

  deg2 = _deg(dst2d)

  sd = jax.ShapeDtypeStruct
  q = OUT // 2
  dinv, h1a, h1b, h1c, h1d = _tc_call(
      _stage_a,
      (sd((N, 1), f32), sd((N, q), f32), sd((N, q), f32), sd((N, q), f32),
       sd((N, q), f32)),
      [_bs((NC, RB, 16), lambda i: (0, i, 0)), _bs((RB, D_IN), _rows),
       _bs((D_IN, HID), _full)],
      [_bs((RB, 1), _rows)] + [_bs((RB, q), _rows)] * 4,
  )(deg2, x, W1n)

  s1a, s1c = _spmm32(h1a, h1c, src2d, dst2d)
  s1b, s1d = _spmm32(h1b, h1d, src2d, dst2d)

  h2l, h2r = _tc_call(
      _stage_b,
      (sd((N, q), f32), sd((N, q), f32)),
      [_bs((RB, q), _rows)] * 4 + [_bs((RB, 1), _rows),
       _bs((1, HID), _full), _bs((HID, OUT), _full)],
      [_bs((RB, q), _rows), _bs((RB, q), _rows)],
  )(s1a, s1b, s1c, s1d, dinv, b1n.reshape(1, HID), W2n)

  s2l, s2r = _spmm32(h2l, h2r, src2d, dst2d)

  x2, h3l, h3r = _tc_call(
      _stage_c,
      (sd((N, OUT), f32), sd((N, OUT // 2), f32), sd((N, OUT // 2), f32)),
      [_bs((RB, OUT // 2), _rows), _bs((RB, OUT // 2), _rows),
       _bs((RB, 1), _rows), _bs((1, OUT), _full), _bs((OUT, OUT), _full)],
      [_bs((RB, OUT), _rows), _bs((RB, OUT // 2), _rows),
       _bs((RB, OUT // 2), _rows)],
  )(s2l, s2r, dinv, b2n.reshape(1, OUT), W2e)

  s3l, s3r = _spmm32(h3l, h3r, src2d, dst2d)

  e = _tc_call(
      _stage_d,
      sd((N, OUT), f32),
      [_bs((RB, OUT // 2), _rows), _bs((RB, OUT // 2), _rows),
       _bs((RB, 1), _rows), _bs((1, OUT), _full)],
      _bs((RB, OUT), _rows),
  )(s3l, s3r, dinv, b2e.reshape(1, OUT))

  return (x2, e)
